# Initial kernel scaffold; baseline (speedup 1.0000x reference)
import jax
import jax.numpy as jnp
from jax.experimental import pallas as pl


def kernel(cls_q, patch_q, cls_bank, patch_bank):
    raise NotImplementedError("write your pallas kernel here")


# speedup vs baseline: 1.4612x; 1.4612x over previous
import functools

import numpy as np
import jax
import jax.numpy as jnp
from jax.experimental import pallas as pl
from jax.experimental.pallas import tpu as pltpu

_INTERPRET = False

K_IMAGE = 5
LAYER_WEIGHTS = (0.5, 0.5)
RESIZE_MASK = 224
ANOMALY_THRESHOLD = 2.0
SCORE_MEAN = 0.3
SCORE_STD = 0.1
_EPS = 1e-12


def _gauss_np(kernel_size=5, sigma=1.0):
    k = np.array([[i * j for j in range(kernel_size)] for i in range(kernel_size)],
                 dtype=np.float32)
    k = np.exp(-k / (2.0 * sigma ** 2))
    return k / k.sum()


def _resize_mat(out_size, in_size):
    R = np.zeros((out_size, in_size), np.float64)
    sc = in_size / out_size
    for o in range(out_size):
        x = (o + 0.5) * sc - 0.5
        lo = int(np.floor(x))
        w = x - lo
        for i, wi in ((lo, 1 - w), (lo + 1, w)):
            R[o, min(max(i, 0), in_size - 1)] += wi
    return R.astype(np.float32)


def _shift_mat(g, a):
    S = np.zeros((g, g), np.float32)
    for p in range(g):
        if 0 <= p + a < g:
            S[p, p + a] = 1.0
    return S


def _conv_resize_mats(g, out_size):
    gk = _gauss_np(5, 1.0)
    R = _resize_mat(out_size, g)
    A = np.zeros((5, out_size, g), np.float32)
    B = np.zeros((5, g, out_size), np.float32)
    for ky in range(5):
        Gk = sum(gk[ky, kx] * _shift_mat(g, kx - 2) for kx in range(5))
        A[ky] = R @ _shift_mat(g, ky - 2)
        B[ky] = Gk.T @ R.T
    return A, B


_A_MAT, _B_MAT = _conv_resize_mats(16, RESIZE_MASK)


def _topk_kernel(cq_ref, cb_ref, idx_ref, *, k, n):
    cq = cq_ref[0]
    cb = cb_ref[0]
    cqn = cq / (jnp.sqrt(jnp.sum(cq * cq, axis=-1, keepdims=True)) + _EPS)
    cbn = cb / (jnp.sqrt(jnp.sum(cb * cb, axis=-1, keepdims=True)) + _EPS)
    sim = jax.lax.dot_general(cqn, cbn, (((1,), (1,)), ((), ())),
                              preferred_element_type=jnp.float32)
    b = sim.shape[0]
    iota = jax.lax.broadcasted_iota(jnp.int32, (b, n), 1)
    cols = []
    for _ in range(k):
        m = jnp.max(sim, axis=1, keepdims=True)
        cur = jnp.min(jnp.where(sim == m, iota, n), axis=1)
        cols.append(cur[:, None])
        sim = jnp.where(iota == cur[:, None], -jnp.inf, sim)
    for _ in range(8 - k):
        cols.append(jnp.zeros((b, 1), jnp.int32))
    idx_ref[0] = jnp.concatenate(cols, axis=1)


def _main_kernel(idx_ref, pq_ref, pb_ref, a_ref, b_ref,
                 amap_ref, score_ref, flag_ref, acc_ref, fused_ref,
                 *, k_image, num_layers, g):
    li = pl.program_id(1)
    kk = pl.program_id(2)

    pq = pq_ref[0, 0]
    pb = pb_ref[0, 0]
    pqn = pq / (jnp.sqrt(jnp.sum(pq * pq, axis=-1, keepdims=True)) + _EPS)
    pbn = pb / (jnp.sqrt(jnp.sum(pb * pb, axis=-1, keepdims=True)) + _EPS)
    s = jax.lax.dot_general(pqn, pbn, (((1,), (1,)), ((), ())),
                            preferred_element_type=jnp.float32)
    m = jnp.max(s.reshape(g, g, s.shape[1]), axis=2)

    @pl.when(kk == 0)
    def _():
        acc_ref[...] = m

    @pl.when(kk > 0)
    def _():
        acc_ref[...] = jnp.maximum(acc_ref[...], m)

    @pl.when((kk == k_image - 1) & (li == 0))
    def _():
        fused_ref[...] = LAYER_WEIGHTS[0] * (1.0 - acc_ref[...])

    @pl.when((kk == k_image - 1) & (li == num_layers - 1))
    def _():
        fused = fused_ref[...] + LAYER_WEIGHTS[num_layers - 1] * (1.0 - acc_ref[...])
        amap = jnp.zeros((amap_ref.shape[1], amap_ref.shape[2]), jnp.float32)
        for ky in range(a_ref.shape[0]):
            t = jax.lax.dot_general(fused, b_ref[ky], (((1,), (0,)), ((), ())),
                                    preferred_element_type=jnp.float32)
            amap = amap + jax.lax.dot_general(
                a_ref[ky], t, (((1,), (0,)), ((), ())),
                preferred_element_type=jnp.float32)
        amap_ref[0] = amap
        sc = jnp.max(amap)
        z = (sc - SCORE_MEAN) / SCORE_STD
        score_ref[...] = jnp.full(score_ref.shape, sc, jnp.float32)
        flag_ref[...] = jnp.full(flag_ref.shape,
                                 (z > ANOMALY_THRESHOLD).astype(jnp.float32))


def kernel(cls_q, patch_q, cls_bank, patch_bank):
    num_layers, b, l, d = patch_q.shape
    n = cls_bank.shape[1]
    g = int(round(l ** 0.5))
    k = K_IMAGE

    idx = pl.pallas_call(
        functools.partial(_topk_kernel, k=k, n=n),
        grid=(num_layers,),
        in_specs=[
            pl.BlockSpec((1, b, d), lambda i: (i, 0, 0)),
            pl.BlockSpec((1, n, d), lambda i: (i, 0, 0)),
        ],
        out_specs=pl.BlockSpec((1, b, 8), lambda i: (i, 0, 0)),
        out_shape=jax.ShapeDtypeStruct((num_layers, b, 8), jnp.int32),
        interpret=_INTERPRET,
    )(cls_q, cls_bank)
    idx_flat = idx[:, :, :k].reshape(-1)

    a_mat = jnp.asarray(_A_MAT)
    b_mat = jnp.asarray(_B_MAT)

    grid = (b, num_layers, k)
    amap, score, flag = pl.pallas_call(
        functools.partial(_main_kernel, k_image=k, num_layers=num_layers, g=g),
        grid_spec=pltpu.PrefetchScalarGridSpec(
            num_scalar_prefetch=1,
            grid=grid,
            in_specs=[
                pl.BlockSpec((1, 1, l, d), lambda bi, li, ki, idx_r: (li, bi, 0, 0)),
                pl.BlockSpec((1, 1, l, d),
                             lambda bi, li, ki, idx_r:
                             (li, idx_r[(li * b + bi) * k + ki], 0, 0)),
                pl.BlockSpec((5, RESIZE_MASK, g), lambda bi, li, ki, idx_r: (0, 0, 0)),
                pl.BlockSpec((5, g, RESIZE_MASK), lambda bi, li, ki, idx_r: (0, 0, 0)),
            ],
            out_specs=[
                pl.BlockSpec((1, RESIZE_MASK, RESIZE_MASK),
                             lambda bi, li, ki, idx_r: (bi, 0, 0)),
                pl.BlockSpec((1, 8, 128), lambda bi, li, ki, idx_r: (bi, 0, 0)),
                pl.BlockSpec((1, 8, 128), lambda bi, li, ki, idx_r: (bi, 0, 0)),
            ],
            scratch_shapes=[
                pltpu.VMEM((g, g), jnp.float32),
                pltpu.VMEM((g, g), jnp.float32),
            ],
        ),
        out_shape=[
            jax.ShapeDtypeStruct((b, RESIZE_MASK, RESIZE_MASK), jnp.float32),
            jax.ShapeDtypeStruct((b, 8, 128), jnp.float32),
            jax.ShapeDtypeStruct((b, 8, 128), jnp.float32),
        ],
        interpret=_INTERPRET,
    )(idx_flat, patch_q, patch_bank, a_mat, b_mat)

    return score[:, 0, 0], flag[:, 0, 0], amap


# speedup vs baseline: 1.4714x; 1.0070x over previous
import functools

import numpy as np
import jax
import jax.numpy as jnp
from jax.experimental import pallas as pl
from jax.experimental.pallas import tpu as pltpu

_INTERPRET = False

K_IMAGE = 5
LAYER_WEIGHTS = (0.5, 0.5)
RESIZE_MASK = 224
ANOMALY_THRESHOLD = 2.0
SCORE_MEAN = 0.3
SCORE_STD = 0.1
_EPS = 1e-12


def _gauss_np(kernel_size=5, sigma=1.0):
    k = np.array([[i * j for j in range(kernel_size)] for i in range(kernel_size)],
                 dtype=np.float32)
    k = np.exp(-k / (2.0 * sigma ** 2))
    return k / k.sum()


def _resize_mat(out_size, in_size):
    R = np.zeros((out_size, in_size), np.float64)
    sc = in_size / out_size
    for o in range(out_size):
        x = (o + 0.5) * sc - 0.5
        lo = int(np.floor(x))
        w = x - lo
        for i, wi in ((lo, 1 - w), (lo + 1, w)):
            R[o, min(max(i, 0), in_size - 1)] += wi
    return R.astype(np.float32)


def _shift_mat(g, a):
    S = np.zeros((g, g), np.float32)
    for p in range(g):
        if 0 <= p + a < g:
            S[p, p + a] = 1.0
    return S


def _conv_resize_mats(g, out_size):
    gk = _gauss_np(5, 1.0)
    R = _resize_mat(out_size, g)
    A = np.zeros((5, out_size, g), np.float32)
    B = np.zeros((5, g, out_size), np.float32)
    for ky in range(5):
        Gk = sum(gk[ky, kx] * _shift_mat(g, kx - 2) for kx in range(5))
        A[ky] = R @ _shift_mat(g, ky - 2)
        B[ky] = Gk.T @ R.T
    return A, B


_A_MAT, _B_MAT = _conv_resize_mats(16, RESIZE_MASK)


def _topk_kernel(cq_ref, cb_ref, idx_ref, *, k, n):
    cq = cq_ref[0]
    cb = cb_ref[0]
    cqn = cq / (jnp.sqrt(jnp.sum(cq * cq, axis=-1, keepdims=True)) + _EPS)
    cbn = cb / (jnp.sqrt(jnp.sum(cb * cb, axis=-1, keepdims=True)) + _EPS)
    sim = jax.lax.dot_general(cqn, cbn, (((1,), (1,)), ((), ())),
                              preferred_element_type=jnp.float32)
    b = sim.shape[0]
    iota = jax.lax.broadcasted_iota(jnp.int32, (b, n), 1)
    cols = []
    for _ in range(k):
        m = jnp.max(sim, axis=1, keepdims=True)
        cur = jnp.min(jnp.where(sim == m, iota, n), axis=1)
        cols.append(cur[:, None])
        sim = jnp.where(iota == cur[:, None], -jnp.inf, sim)
    for _ in range(8 - k):
        cols.append(jnp.zeros((b, 1), jnp.int32))
    idx_ref[0] = jnp.concatenate(cols, axis=1)


def _main_kernel(idx_ref, pq_ref, pb_ref, a_ref, b_ref,
                 amap_ref, score_ref, flag_ref, acc_ref, fused_ref,
                 *, k_image, num_layers, g):
    li = pl.program_id(1)
    kk = pl.program_id(2)

    pq = pq_ref[0, 0]
    pb = pb_ref[0, 0]
    pqn = (pq / (jnp.sqrt(jnp.sum(pq * pq, axis=-1, keepdims=True)) + _EPS)
           ).astype(jnp.bfloat16)
    pbn = (pb / (jnp.sqrt(jnp.sum(pb * pb, axis=-1, keepdims=True)) + _EPS)
           ).astype(jnp.bfloat16)
    s = jax.lax.dot_general(pqn, pbn, (((1,), (1,)), ((), ())),
                            preferred_element_type=jnp.float32)
    m = jnp.max(s.reshape(g, g, s.shape[1]), axis=2)

    @pl.when(kk == 0)
    def _():
        acc_ref[...] = m

    @pl.when(kk > 0)
    def _():
        acc_ref[...] = jnp.maximum(acc_ref[...], m)

    @pl.when((kk == k_image - 1) & (li == 0))
    def _():
        fused_ref[...] = LAYER_WEIGHTS[0] * (1.0 - acc_ref[...])

    @pl.when((kk == k_image - 1) & (li == num_layers - 1))
    def _():
        fused = fused_ref[...] + LAYER_WEIGHTS[num_layers - 1] * (1.0 - acc_ref[...])
        amap = jnp.zeros((amap_ref.shape[1], amap_ref.shape[2]), jnp.float32)
        for ky in range(a_ref.shape[0]):
            t = jax.lax.dot_general(fused, b_ref[ky], (((1,), (0,)), ((), ())),
                                    preferred_element_type=jnp.float32)
            amap = amap + jax.lax.dot_general(
                a_ref[ky], t, (((1,), (0,)), ((), ())),
                preferred_element_type=jnp.float32)
        amap_ref[0] = amap
        sc = jnp.max(amap)
        z = (sc - SCORE_MEAN) / SCORE_STD
        score_ref[...] = jnp.full(score_ref.shape, sc, jnp.float32)
        flag_ref[...] = jnp.full(flag_ref.shape,
                                 (z > ANOMALY_THRESHOLD).astype(jnp.float32))


def kernel(cls_q, patch_q, cls_bank, patch_bank):
    num_layers, b, l, d = patch_q.shape
    n = cls_bank.shape[1]
    g = int(round(l ** 0.5))
    k = K_IMAGE

    idx = pl.pallas_call(
        functools.partial(_topk_kernel, k=k, n=n),
        grid=(num_layers,),
        in_specs=[
            pl.BlockSpec((1, b, d), lambda i: (i, 0, 0)),
            pl.BlockSpec((1, n, d), lambda i: (i, 0, 0)),
        ],
        out_specs=pl.BlockSpec((1, b, 8), lambda i: (i, 0, 0)),
        out_shape=jax.ShapeDtypeStruct((num_layers, b, 8), jnp.int32),
        interpret=_INTERPRET,
    )(cls_q, cls_bank)
    idx_flat = idx[:, :, :k].reshape(-1)

    a_mat = jnp.asarray(_A_MAT)
    b_mat = jnp.asarray(_B_MAT)

    grid = (b, num_layers, k)
    amap, score, flag = pl.pallas_call(
        functools.partial(_main_kernel, k_image=k, num_layers=num_layers, g=g),
        grid_spec=pltpu.PrefetchScalarGridSpec(
            num_scalar_prefetch=1,
            grid=grid,
            in_specs=[
                pl.BlockSpec((1, 1, l, d), lambda bi, li, ki, idx_r: (li, bi, 0, 0)),
                pl.BlockSpec((1, 1, l, d),
                             lambda bi, li, ki, idx_r:
                             (li, idx_r[(li * b + bi) * k + ki], 0, 0)),
                pl.BlockSpec((5, RESIZE_MASK, g), lambda bi, li, ki, idx_r: (0, 0, 0)),
                pl.BlockSpec((5, g, RESIZE_MASK), lambda bi, li, ki, idx_r: (0, 0, 0)),
            ],
            out_specs=[
                pl.BlockSpec((1, RESIZE_MASK, RESIZE_MASK),
                             lambda bi, li, ki, idx_r: (bi, 0, 0)),
                pl.BlockSpec((1, 8, 128), lambda bi, li, ki, idx_r: (bi, 0, 0)),
                pl.BlockSpec((1, 8, 128), lambda bi, li, ki, idx_r: (bi, 0, 0)),
            ],
            scratch_shapes=[
                pltpu.VMEM((g, g), jnp.float32),
                pltpu.VMEM((g, g), jnp.float32),
            ],
        ),
        out_shape=[
            jax.ShapeDtypeStruct((b, RESIZE_MASK, RESIZE_MASK), jnp.float32),
            jax.ShapeDtypeStruct((b, 8, 128), jnp.float32),
            jax.ShapeDtypeStruct((b, 8, 128), jnp.float32),
        ],
        interpret=_INTERPRET,
    )(idx_flat, patch_q, patch_bank, a_mat, b_mat)

    return score[:, 0, 0], flag[:, 0, 0], amap


# speedup vs baseline: 3.3056x; 2.2465x over previous
import functools

import numpy as np
import jax
import jax.numpy as jnp
from jax import lax
from jax.experimental import pallas as pl
from jax.experimental.pallas import tpu as pltpu
from jax.experimental.pallas import tpu_sc as plsc

K_IMAGE = 5
LAYER_WEIGHTS = (0.5, 0.5)
RESIZE_MASK = 224
ANOMALY_THRESHOLD = 2.0
SCORE_MEAN = 0.3
SCORE_STD = 0.1
_EPS = 1e-12


def _gauss_np(kernel_size=5, sigma=1.0):
    k = np.array([[i * j for j in range(kernel_size)] for i in range(kernel_size)],
                 dtype=np.float32)
    k = np.exp(-k / (2.0 * sigma ** 2))
    return k / k.sum()


def _resize_mat(out_size, in_size):
    R = np.zeros((out_size, in_size), np.float64)
    sc = in_size / out_size
    for o in range(out_size):
        x = (o + 0.5) * sc - 0.5
        lo = int(np.floor(x))
        w = x - lo
        for i, wi in ((lo, 1 - w), (lo + 1, w)):
            R[o, min(max(i, 0), in_size - 1)] += wi
    return R.astype(np.float32)


def _shift_mat(g, a):
    S = np.zeros((g, g), np.float32)
    for p in range(g):
        if 0 <= p + a < g:
            S[p, p + a] = 1.0
    return S


def _conv_resize_mats(g, out_size):
    gk = _gauss_np(5, 1.0)
    R = _resize_mat(out_size, g)
    A = np.zeros((5, out_size, g), np.float32)
    B = np.zeros((5, g, out_size), np.float32)
    for ky in range(5):
        Gk = sum(gk[ky, kx] * _shift_mat(g, kx - 2) for kx in range(5))
        A[ky] = R @ _shift_mat(g, ky - 2)
        B[ky] = Gk.T @ R.T
    return A, B


_A_MAT, _B_MAT = _conv_resize_mats(16, RESIZE_MASK)


def _sim_kernel(cq_ref, cb_ref, sim_ref, *, n):
    for li in range(cq_ref.shape[0]):
        cq = cq_ref[li]
        cb = cb_ref[li]
        cqn = cq / (jnp.sqrt(jnp.sum(cq * cq, axis=-1, keepdims=True)) + _EPS)
        cbn = cb / (jnp.sqrt(jnp.sum(cb * cb, axis=-1, keepdims=True)) + _EPS)
        sim = jax.lax.dot_general(cqn, cbn, (((1,), (1,)), ((), ())),
                                  preferred_element_type=jnp.float32)
        b = sim.shape[0]
        pad = jnp.full((b, sim_ref.shape[2] - n), -jnp.inf, jnp.float32)
        sim_ref[li] = jnp.concatenate([sim, pad], axis=1)


_SC_NC, _SC_NS, _SC_L = 2, 16, 16


def _sc_topk_body(sim_hbm, out_hbm, row_v, idx_v, *, k, nchunks):
    wid = lax.axis_index("s") * _SC_NC + lax.axis_index("c")
    pltpu.sync_copy(sim_hbm.at[wid], row_v)
    regs = [row_v[pl.ds(c * _SC_L, _SC_L)] for c in range(nchunks)]
    iota = lax.iota(jnp.int32, _SC_L)
    neg = jnp.full((_SC_L,), -jnp.inf, jnp.float32)
    idxv = jnp.zeros((_SC_L,), jnp.int32)
    for j in range(k):
        vm = regs[0]
        for c in range(1, nchunks):
            vm = jnp.maximum(vm, regs[c])
        for sh in (8, 4, 2, 1):
            vm = jnp.maximum(vm, vm.at[iota ^ sh].get(mode='promise_in_bounds'))
        best = vm
        cand = None
        for c in range(nchunks):
            cc = jnp.where(regs[c] == best, iota + c * _SC_L,
                           jnp.full((_SC_L,), 2 ** 30, jnp.int32))
            cand = cc if cand is None else jnp.minimum(cand, cc)
        for sh in (8, 4, 2, 1):
            cand = jnp.minimum(cand,
                               cand.at[iota ^ sh].get(mode='promise_in_bounds'))
        g = cand
        idxv = jnp.where(iota == j, g, idxv)
        for c in range(nchunks):
            regs[c] = jnp.where(iota + c * _SC_L == g, neg, regs[c])
    idx_v[...] = idxv
    pltpu.sync_copy(idx_v, out_hbm.at[wid])


def _norm_bf16(x):
    return (x / (jnp.sqrt(jnp.sum(x * x, axis=-1, keepdims=True)) + _EPS)
            ).astype(jnp.bfloat16)


def _main_kernel(idx_ref, *refs, k_image, num_layers, g):
    bi = pl.program_id(0)
    pq_refs = refs[:num_layers]
    pb_refs = refs[num_layers:num_layers + num_layers * k_image]
    a_ref, b_ref, amap_ref, score_ref, flag_ref = refs[num_layers + num_layers * k_image:]

    fused = jnp.zeros((g, g), jnp.float32)
    for li in range(num_layers):
        pqn = _norm_bf16(pq_refs[li][0])
        ms = None
        for ki in range(k_image):
            pbn = _norm_bf16(pb_refs[li * k_image + ki][0])
            s = jax.lax.dot_general(pqn, pbn, (((1,), (1,)), ((), ())),
                                    preferred_element_type=jnp.float32)
            m = jnp.max(s.reshape(g, g, s.shape[1]), axis=2)
            ms = m if ms is None else jnp.maximum(ms, m)
        fused = fused + LAYER_WEIGHTS[li] * (1.0 - ms)

    amap = jnp.zeros((amap_ref.shape[1], amap_ref.shape[2]), jnp.float32)
    for ky in range(a_ref.shape[0]):
        t = jax.lax.dot_general(fused, b_ref[ky], (((1,), (0,)), ((), ())),
                                preferred_element_type=jnp.float32)
        amap = amap + jax.lax.dot_general(
            a_ref[ky], t, (((1,), (0,)), ((), ())),
            preferred_element_type=jnp.float32)
    amap_ref[0] = amap
    sc = jnp.max(amap)
    z = (sc - SCORE_MEAN) / SCORE_STD
    lane = jax.lax.broadcasted_iota(jnp.int32, score_ref.shape, 1)
    score_ref[...] = jnp.where(lane == bi, sc, score_ref[...])
    flag_ref[...] = jnp.where(lane == bi,
                              (z > ANOMALY_THRESHOLD).astype(jnp.float32),
                              flag_ref[...])


def kernel(cls_q, patch_q, cls_bank, patch_bank):
    num_layers, b, l, d = patch_q.shape
    n = cls_bank.shape[1]
    g = int(round(l ** 0.5))
    k = K_IMAGE

    lanes = 256
    sim = pl.pallas_call(
        functools.partial(_sim_kernel, n=n),
        out_shape=jax.ShapeDtypeStruct((num_layers, b, lanes), jnp.float32),
    )(cls_q, cls_bank)
    sim2d = sim.reshape(num_layers * b, lanes)

    sc_topk = functools.partial(
        pl.kernel,
        out_type=jax.ShapeDtypeStruct((num_layers * b, _SC_L), jnp.int32),
        mesh=plsc.VectorSubcoreMesh(core_axis_name="c", subcore_axis_name="s"),
        scratch_types=[
            pltpu.VMEM((lanes,), jnp.float32),
            pltpu.VMEM((_SC_L,), jnp.int32),
        ],
    )(functools.partial(_sc_topk_body, k=k, nchunks=lanes // _SC_L))
    idx32 = sc_topk(sim2d)
    idx_flat = idx32.reshape(-1)

    a_mat = jnp.asarray(_A_MAT)
    b_mat = jnp.asarray(_B_MAT)

    pq_flat = patch_q.reshape(num_layers * b, l, d)
    pb_flat = patch_bank.reshape(num_layers * n, l, d)

    pq_specs = [
        pl.BlockSpec((1, l, d), lambda bi, idx_r, li=li: (li * b + bi, 0, 0))
        for li in range(num_layers)
    ]
    pb_specs = [
        pl.BlockSpec((1, l, d),
                     lambda bi, idx_r, li=li, ki=ki:
                     (li * n + idx_r[(li * b + bi) * _SC_L + ki], 0, 0))
        for li in range(num_layers) for ki in range(k)
    ]

    amap, score, flag = pl.pallas_call(
        functools.partial(_main_kernel, k_image=k, num_layers=num_layers, g=g),
        grid_spec=pltpu.PrefetchScalarGridSpec(
            num_scalar_prefetch=1,
            grid=(b,),
            in_specs=pq_specs + pb_specs + [
                pl.BlockSpec((5, RESIZE_MASK, g), lambda bi, idx_r: (0, 0, 0)),
                pl.BlockSpec((5, g, RESIZE_MASK), lambda bi, idx_r: (0, 0, 0)),
            ],
            out_specs=[
                pl.BlockSpec((1, RESIZE_MASK, RESIZE_MASK),
                             lambda bi, idx_r: (bi, 0, 0)),
                pl.BlockSpec((1, b), lambda bi, idx_r: (0, 0)),
                pl.BlockSpec((1, b), lambda bi, idx_r: (0, 0)),
            ],
        ),
        out_shape=[
            jax.ShapeDtypeStruct((b, RESIZE_MASK, RESIZE_MASK), jnp.float32),
            jax.ShapeDtypeStruct((1, b), jnp.float32),
            jax.ShapeDtypeStruct((1, b), jnp.float32),
        ],
    )(idx_flat, *([pq_flat] * num_layers), *([pb_flat] * (num_layers * k)),
      a_mat, b_mat)

    return score.reshape(b), flag.reshape(b), amap
